# Initial kernel scaffold; baseline (speedup 1.0000x reference)
import jax
import jax.numpy as jnp
from jax.experimental import pallas as pl


def kernel(x, edge_index, c1_pre_W, c1_pre_b, c1_post_W, c1_post_b, c1_lin_W, c1_lin_b, c2_pre_W, c2_pre_b, c2_post_W, c2_post_b, c2_lin_W, c2_lin_b, out_W, out_b):
    raise NotImplementedError("write your pallas kernel here")


# speedup vs baseline: 5.6374x; 5.6374x over previous
import functools

import jax
import jax.numpy as jnp
from jax import lax
from jax.experimental import pallas as pl
from jax.experimental.pallas import tpu as pltpu
from jax.experimental.pallas import tpu_sc as plsc

NP = 960
C = 128
NEG = -3.0e38
POS = 3.0e38


def _make_edge_stats(F_pad, NWIN, E_alloc, n_rows):
    NV = F_pad // 16
    N_pad = NWIN * NP
    mesh = plsc.VectorSubcoreMesh(
        core_axis_name="c", subcore_axis_name="s", num_cores=2, num_subcores=16)

    out_t = tuple(jax.ShapeDtypeStruct((N_pad * F_pad,), jnp.float32)
                  for _ in range(4))

    @functools.partial(
        pl.kernel,
        out_type=out_t,
        mesh=mesh,
        compiler_params=pltpu.CompilerParams(
            needs_layout_passes=False, use_tc_tiling_on_sc=False),
        scratch_types=[
            pltpu.VMEM((NP * F_pad,), jnp.float32),
            pltpu.VMEM((NP * F_pad,), jnp.float32),
            pltpu.VMEM((NP * F_pad,), jnp.float32),
            pltpu.VMEM((NP * F_pad,), jnp.float32),
            pltpu.VMEM((C,), jnp.int32),
            pltpu.VMEM((C + 16,), jnp.int32),
            pltpu.VMEM((C, F_pad), jnp.float32),
            pltpu.VMEM((128,), jnp.int32),
            pltpu.VMEM((64,), jnp.int32),
            pltpu.SemaphoreType.DMA,
        ],
    )
    def kern(t_h, src_h, dst_h, weo_h, wcut_h, z_h, neg_h, pos_h,
             S_h, Q_h, M_h, N_h,
             accS, accQ, accM, accN, sv, dv, rows, weo_v, wcut_v, sem):
        w = lax.axis_index("c") * 16 + lax.axis_index("s")
        pltpu.sync_copy(weo_h, weo_v)
        pltpu.sync_copy(wcut_h, wcut_v)
        lanes = lax.iota(jnp.int32, 16)
        zero16 = jnp.zeros((16,), jnp.float32)
        neg16 = jnp.full((16,), NEG, jnp.float32)
        pos16 = jnp.full((16,), POS, jnp.float32)

        def do_flush(j, cur, rcnt, rs, rq, rm, rn):
            doff = jnp.clip(cur - j * NP, 0, NP - 1)
            rowbase = jnp.full((16,), doff * F_pad, jnp.int32) + lanes
            cntf = rcnt.astype(jnp.float32)
            rs = list(rs)
            rs[NV - 1] = jnp.where(lanes == 15, jnp.full((16,), cntf),
                                   rs[NV - 1])
            for i in range(NV):
                idx = rowbase + 16 * i
                plsc.store_scatter(accS, [idx], rs[i])
                plsc.store_scatter(accQ, [idx], rq[i])
                plsc.store_scatter(accM, [idx], rm[i])
                plsc.store_scatter(accN, [idx], rn[i])

        def win_body(j, _):
            ev = weo_v[pl.ds(j, 16)]
            e0 = ev[0]
            e1 = ev[1]
            base = (e0 // 8) * 8
            nch = (e1 - base + (C - 1)) // C
            pltpu.sync_copy(z_h, accS)
            pltpu.sync_copy(z_h, accQ)
            pltpu.sync_copy(neg_h, accM)
            pltpu.sync_copy(pos_h, accN)

            def chunk_body(k, regs):
                off = base + k * C
                pltpu.sync_copy(src_h.at[pl.ds(off, C)], sv)
                pltpu.sync_copy(dst_h.at[pl.ds(off, C)], dv.at[pl.ds(0, C)])
                pltpu.async_copy(t_h.at[sv], rows, sem).wait()
                lo = jnp.maximum(e0 - off, 0)
                hi = jnp.minimum(e1 - off, C)

                def edge_body(c, regs):
                    (rs, rq, rm, rn, cur, rcnt) = regs
                    d = dv[pl.ds(c, 16)][0]
                    v = [rows[c, pl.ds(16 * i, 16)] for i in range(NV)]
                    chg = d != cur

                    @pl.when(chg)
                    def _():
                        do_flush(j, cur, rcnt, rs, rq, rm, rn)

                    m = jnp.full((16,), chg)
                    rs2 = [jnp.where(m, v[i], rs[i] + v[i]) for i in range(NV)]
                    rq2 = [jnp.where(m, v[i] * v[i], rq[i] + v[i] * v[i])
                           for i in range(NV)]
                    rm2 = [jnp.maximum(jnp.where(m, neg16, rm[i]), v[i])
                           for i in range(NV)]
                    rn2 = [jnp.minimum(jnp.where(m, pos16, rn[i]), v[i])
                           for i in range(NV)]
                    rcnt2 = jnp.where(chg, jnp.int32(1), rcnt + 1)
                    return (rs2, rq2, rm2, rn2, d, rcnt2)

                return lax.fori_loop(lo, hi, edge_body, regs)

            init = ([zero16] * NV, [zero16] * NV, [neg16] * NV, [pos16] * NV,
                    jnp.int32(-1), jnp.int32(0))
            (rs, rq, rm, rn, cur, rcnt) = lax.fori_loop(0, nch, chunk_body, init)

            @pl.when(cur >= 0)
            def _():
                do_flush(j, cur, rcnt, rs, rq, rm, rn)

            pltpu.sync_copy(accS, S_h.at[pl.ds(j * (NP * F_pad), NP * F_pad)])
            pltpu.sync_copy(accQ, Q_h.at[pl.ds(j * (NP * F_pad), NP * F_pad)])
            pltpu.sync_copy(accM, M_h.at[pl.ds(j * (NP * F_pad), NP * F_pad)])
            pltpu.sync_copy(accN, N_h.at[pl.ds(j * (NP * F_pad), NP * F_pad)])
            return 0

        wv = wcut_v[pl.ds(w, 16)]
        lax.fori_loop(wv[0], wv[1], win_body, 0)

    return kern


def _dense_post(x, a, S, Q, Mx, Mn, cnt, has, denom, avg_lin, avg_log,
                postW, postb, linW, linb):
    meanb = S / denom
    mean = jnp.where(has, a + meanb, 0.0)
    mx = jnp.where(has, a + Mx, 0.0)
    mn = jnp.where(has, a + Mn, 0.0)
    var = jnp.where(has, jax.nn.relu(Q / denom - meanb * meanb), 0.0)
    std = jnp.sqrt(var + 1e-5)
    base = jnp.concatenate([mean, mn, mx, std], axis=-1)
    deg = jnp.clip(cnt, 1.0)[:, None]
    s_amp = jnp.log(deg + 1.0) / avg_log
    out = jnp.concatenate(
        [x, base, base * s_amp, base / s_amp, base * (deg / avg_lin)], axis=-1)
    out = out @ postW + postb
    return out @ linW + linb


def kernel(x, edge_index, c1_pre_W, c1_pre_b, c1_post_W, c1_post_b, c1_lin_W, c1_lin_b, c2_pre_W, c2_pre_b, c2_post_W, c2_post_b, c2_lin_W, c2_lin_b, out_W, out_b):
    n = x.shape[0]
    E = edge_index.shape[1]
    NWIN = (n + NP - 1) // NP
    N_pad = NWIN * NP
    E_alloc = E + 2 * C

    src = edge_index[0]
    dst = edge_index[1]
    perm = jnp.argsort(dst)
    sdst = dst[perm]
    ssrc = src[perm]
    weo = jnp.searchsorted(sdst, jnp.arange(NWIN + 1) * NP).astype(jnp.int32)
    weo_p = jnp.zeros((128,), jnp.int32).at[: NWIN + 1].set(weo)
    targets = (jnp.arange(33) * (E // 32)).astype(jnp.int32)
    wcut = jnp.searchsorted(weo, targets).astype(jnp.int32)
    wcut = wcut.at[0].set(0).at[32].set(NWIN)
    wcut_p = jnp.zeros((64,), jnp.int32).at[:33].set(
        jnp.minimum(wcut, NWIN))
    ssrc_p = jnp.zeros((E_alloc,), jnp.int32).at[:E].set(ssrc)
    sdst_p = jnp.zeros((E_alloc,), jnp.int32).at[:E].set(sdst)

    def run_conv(h, F, F_pad, preW, preb, postW, postb, linW, linb,
                 cnt=None, has=None, denom=None, avg_lin=None, avg_log=None):
        a = h @ preW[:F] + preb
        t = h @ preW[F:]
        t_pad = jnp.zeros((n, F_pad), jnp.float32).at[:, :F].set(t)
        z = jnp.zeros((NP * F_pad,), jnp.float32)
        ng = jnp.full((NP * F_pad,), NEG, jnp.float32)
        ps = jnp.full((NP * F_pad,), POS, jnp.float32)
        kern = _make_edge_stats(F_pad, NWIN, E_alloc, n)
        S, Q, Mx, Mn = kern(t_pad, ssrc_p, sdst_p, weo_p, wcut_p, z, ng, ps)
        S = S.reshape(N_pad, F_pad)
        Q = Q.reshape(N_pad, F_pad)
        Mx = Mx.reshape(N_pad, F_pad)
        Mn = Mn.reshape(N_pad, F_pad)
        if cnt is None:
            cnt = S[:n, F_pad - 1]
            has = (cnt > 0)[:, None]
            denom = jnp.clip(cnt, 1.0)[:, None]
            avg_lin = jnp.mean(cnt)
            avg_log = jnp.mean(jnp.log(cnt + 1.0))
        out = _dense_post(h, a, S[:n, :F], Q[:n, :F], Mx[:n, :F], Mn[:n, :F],
                          cnt, has, denom, avg_lin, avg_log,
                          postW, postb, linW, linb)
        return out, cnt, has, denom, avg_lin, avg_log

    h1, cnt, has, denom, avg_lin, avg_log = run_conv(
        x, 3, 16, c1_pre_W, c1_pre_b, c1_post_W, c1_post_b, c1_lin_W, c1_lin_b)
    h1 = jax.nn.relu(h1)
    h2, *_ = run_conv(h1, 20, 32, c2_pre_W, c2_pre_b, c2_post_W, c2_post_b,
                      c2_lin_W, c2_lin_b, cnt, has, denom, avg_lin, avg_log)
    h2 = jax.nn.relu(h2)
    return h2 @ out_W + out_b


# speedup vs baseline: 8.5356x; 1.5141x over previous
import functools

import jax
import jax.numpy as jnp
from jax import lax
from jax.experimental import pallas as pl
from jax.experimental.pallas import tpu as pltpu
from jax.experimental.pallas import tpu_sc as plsc

C = 256
NQ = C // 128
NEG = -3.0e38
POS = 3.0e38


def _make_edge_stats(F_pad, NP, NWIN, E_alloc, n_rows):
    NV = F_pad // 16
    N_pad = NWIN * NP
    AW = NP * F_pad
    mesh = plsc.VectorSubcoreMesh(
        core_axis_name="c", subcore_axis_name="s", num_cores=2, num_subcores=16)

    out_t = tuple(jax.ShapeDtypeStruct((N_pad * F_pad,), jnp.float32)
                  for _ in range(4))

    @functools.partial(
        pl.kernel,
        out_type=out_t,
        mesh=mesh,
        compiler_params=pltpu.CompilerParams(
            needs_layout_passes=False, use_tc_tiling_on_sc=False),
        scratch_types=[
            pltpu.VMEM((AW,), jnp.float32),
            pltpu.VMEM((AW,), jnp.float32),
            pltpu.VMEM((AW,), jnp.float32),
            pltpu.VMEM((AW,), jnp.float32),
            pltpu.VMEM((C,), jnp.int32),
            pltpu.VMEM((C,), jnp.int32),
            pltpu.VMEM((C,), jnp.int32),
            pltpu.VMEM((C,), jnp.int32),
            pltpu.VMEM((C, F_pad), jnp.float32),
            pltpu.VMEM((C, F_pad), jnp.float32),
            pltpu.VMEM((160,), jnp.int32),
            pltpu.VMEM((64,), jnp.int32),
            pltpu.SemaphoreType.DMA,
            pltpu.SemaphoreType.DMA,
            pltpu.SemaphoreType.DMA,
            pltpu.SemaphoreType.DMA,
        ],
    )
    def kern(t_h, src_h, dst_h, weo_h, wcut_h, z_h, neg_h, pos_h,
             S_h, Q_h, M_h, N_h,
             accS, accQ, accM, accN, svA, svB, dvA, dvB, rowsA, rowsB,
             weo_v, wcut_v, semiA, semiB, semgA, semgB):
        w = lax.axis_index("c") * 16 + lax.axis_index("s")
        pltpu.sync_copy(weo_h, weo_v)
        pltpu.sync_copy(wcut_h, wcut_v)
        lanes = lax.iota(jnp.int32, 16)
        zero16 = jnp.zeros((16,), jnp.float32)
        neg16 = jnp.full((16,), NEG, jnp.float32)
        pos16 = jnp.full((16,), POS, jnp.float32)
        bufs = ((svA, dvA, rowsA, semiA, semgA),
                (svB, dvB, rowsB, semiB, semgB))

        def do_flush(j, cur, rcnt, rs, rq, rm, rn):
            doff = jnp.clip(cur - j * NP, 0, NP - 1)
            rowbase = jnp.full((16,), doff * F_pad, jnp.int32) + lanes
            cntf = rcnt.astype(jnp.float32)
            rs = list(rs)
            rs[NV - 1] = jnp.where(lanes == 15, jnp.full((16,), cntf),
                                   rs[NV - 1])
            for i in range(NV):
                idx = rowbase + 16 * i
                plsc.store_scatter(accS, [idx], rs[i])
                plsc.store_scatter(accQ, [idx], rq[i])
                plsc.store_scatter(accM, [idx], rm[i])
                plsc.store_scatter(accN, [idx], rn[i])

        def win_body(j, _):
            ev = weo_v[pl.ds(j, 16)]
            e0 = ev[0]
            e1 = ev[1]
            base = (e0 // 8) * 8
            nch = (e1 - base + (C - 1)) // C
            nch2 = (nch + 1) // 2
            pltpu.sync_copy(z_h, accS)
            pltpu.sync_copy(z_h, accQ)
            pltpu.sync_copy(neg_h, accM)
            pltpu.sync_copy(pos_h, accN)

            def start_idx(k, b):
                sv, dv, _, semi, _ = bufs[b]
                off = base + k * C
                pltpu.make_async_copy(src_h.at[pl.ds(off, C)], sv, semi).start()
                pltpu.make_async_copy(dst_h.at[pl.ds(off, C)], dv, semi).start()

            def wait_idx(k, b):
                sv, dv, _, semi, _ = bufs[b]
                off = base + k * C
                pltpu.make_async_copy(src_h.at[pl.ds(off, C)], sv, semi).wait()
                pltpu.make_async_copy(dst_h.at[pl.ds(off, C)], dv, semi).wait()

            def start_gather(b):
                sv, _, rows, _, semg = bufs[b]
                for q in range(NQ):
                    pltpu.make_async_copy(
                        t_h.at[sv.at[pl.ds(q * 128, 128)]],
                        rows.at[pl.ds(q * 128, 128)], semg).start()

            def wait_gather(b):
                sv, _, rows, _, semg = bufs[b]
                for q in range(NQ):
                    pltpu.make_async_copy(
                        t_h.at[sv.at[pl.ds(q * 128, 128)]],
                        rows.at[pl.ds(q * 128, 128)], semg).wait()

            start_idx(0, 0)
            wait_idx(0, 0)
            start_gather(0)
            start_idx(1, 1)

            def compute_chunk(k, b, regs):
                _, dv, rows, _, _ = bufs[b]
                off = base + k * C

                def body16(g, regs):
                    (rs, rq, rm, rn, cur, rcnt) = regs
                    goff = g * 16
                    dvec = dv[pl.ds(goff, 16)]
                    for u in range(16):
                        pos = off + goff + u
                        d = dvec[u]
                        valid = jnp.logical_and(pos >= e0, pos < e1)
                        validv = jnp.full((16,), valid)
                        chg = jnp.logical_and(d != cur, valid)

                        @pl.when(chg)
                        def _(cur=cur, rcnt=rcnt, rs=rs, rq=rq, rm=rm, rn=rn):
                            do_flush(j, cur, rcnt, rs, rq, rm, rn)

                        mm = jnp.full((16,), chg)
                        v = [rows[goff + u, pl.ds(16 * i, 16)]
                             for i in range(NV)]
                        vS = [jnp.where(validv, vi, zero16) for vi in v]
                        vM = [jnp.where(validv, vi, neg16) for vi in v]
                        vN = [jnp.where(validv, vi, pos16) for vi in v]
                        rs = [jnp.where(mm, vS[i], rs[i] + vS[i])
                              for i in range(NV)]
                        rq = [jnp.where(mm, vS[i] * vS[i],
                                        rq[i] + vS[i] * vS[i])
                              for i in range(NV)]
                        rm = [jnp.maximum(jnp.where(mm, neg16, rm[i]), vM[i])
                              for i in range(NV)]
                        rn = [jnp.minimum(jnp.where(mm, pos16, rn[i]), vN[i])
                              for i in range(NV)]
                        cur = jnp.where(chg, d, cur)
                        rcnt = (jnp.where(chg, jnp.int32(0), rcnt)
                                + valid.astype(jnp.int32))
                    return (rs, rq, rm, rn, cur, rcnt)

                return lax.fori_loop(0, C // 16, body16, regs)

            def chunk2_body(k2, regs):
                k = k2 * 2
                wait_gather(0)
                wait_idx(k + 1, 1)
                start_gather(1)
                regs = compute_chunk(k, 0, regs)
                start_idx(k + 2, 0)
                wait_gather(1)
                wait_idx(k + 2, 0)
                start_gather(0)
                regs = compute_chunk(k + 1, 1, regs)
                start_idx(k + 3, 1)
                return regs

            init = ([zero16] * NV, [zero16] * NV, [neg16] * NV, [pos16] * NV,
                    jnp.int32(-1), jnp.int32(0))
            (rs, rq, rm, rn, cur, rcnt) = lax.fori_loop(
                0, nch2, chunk2_body, init)

            wait_gather(0)
            wait_idx(2 * nch2 + 1, 1)

            @pl.when(cur >= 0)
            def _():
                do_flush(j, cur, rcnt, rs, rq, rm, rn)

            pltpu.sync_copy(accS, S_h.at[pl.ds(j * AW, AW)])
            pltpu.sync_copy(accQ, Q_h.at[pl.ds(j * AW, AW)])
            pltpu.sync_copy(accM, M_h.at[pl.ds(j * AW, AW)])
            pltpu.sync_copy(accN, N_h.at[pl.ds(j * AW, AW)])
            return 0

        wv = wcut_v[pl.ds(w, 16)]
        lax.fori_loop(wv[0], wv[1], win_body, 0)

    return kern


def _dense_post(x, a, S, Q, Mx, Mn, cnt, has, denom, avg_lin, avg_log,
                postW, postb, linW, linb):
    meanb = S / denom
    mean = jnp.where(has, a + meanb, 0.0)
    mx = jnp.where(has, a + Mx, 0.0)
    mn = jnp.where(has, a + Mn, 0.0)
    var = jnp.where(has, jax.nn.relu(Q / denom - meanb * meanb), 0.0)
    std = jnp.sqrt(var + 1e-5)
    base = jnp.concatenate([mean, mn, mx, std], axis=-1)
    deg = jnp.clip(cnt, 1.0)[:, None]
    s_amp = jnp.log(deg + 1.0) / avg_log
    out = jnp.concatenate(
        [x, base, base * s_amp, base / s_amp, base * (deg / avg_lin)], axis=-1)
    out = out @ postW + postb
    return out @ linW + linb


def kernel(x, edge_index, c1_pre_W, c1_pre_b, c1_post_W, c1_post_b, c1_lin_W, c1_lin_b, c2_pre_W, c2_pre_b, c2_post_W, c2_post_b, c2_lin_W, c2_lin_b, out_W, out_b):
    n = x.shape[0]
    E = edge_index.shape[1]
    E_alloc = E + 4 * C

    src = edge_index[0]
    dst = edge_index[1]
    perm = jnp.argsort(dst)
    sdst = dst[perm]
    ssrc = src[perm]
    ssrc_p = jnp.zeros((E_alloc,), jnp.int32).at[:E].set(ssrc)
    sdst_p = jnp.zeros((E_alloc,), jnp.int32).at[:E].set(sdst)

    def make_offsets(NP):
        NWIN = (n + NP - 1) // NP
        weo = jnp.searchsorted(
            sdst, jnp.arange(NWIN + 1) * NP).astype(jnp.int32)
        weo_p = jnp.zeros((160,), jnp.int32).at[: NWIN + 1].set(weo)
        targets = (jnp.arange(33) * (E // 32)).astype(jnp.int32)
        wcut = jnp.searchsorted(weo, targets).astype(jnp.int32)
        wcut = wcut.at[0].set(0).at[32].set(NWIN)
        wcut_p = jnp.zeros((64,), jnp.int32).at[:33].set(
            jnp.minimum(wcut, NWIN))
        return NWIN, weo_p, wcut_p

    NP1, NP2 = 1824, 864
    offs = {NP1: make_offsets(NP1), NP2: make_offsets(NP2)}

    def run_conv(h, F, F_pad, NP, preW, preb, postW, postb, linW, linb,
                 cnt=None, has=None, denom=None, avg_lin=None, avg_log=None):
        NWIN, weo_p, wcut_p = offs[NP]
        N_pad = NWIN * NP
        a = h @ preW[:F] + preb
        t = h @ preW[F:]
        t_pad = jnp.zeros((n, F_pad), jnp.float32).at[:, :F].set(t)
        z = jnp.zeros((NP * F_pad,), jnp.float32)
        ng = jnp.full((NP * F_pad,), NEG, jnp.float32)
        ps = jnp.full((NP * F_pad,), POS, jnp.float32)
        kern = _make_edge_stats(F_pad, NP, NWIN, E_alloc, n)
        S, Q, Mx, Mn = kern(t_pad, ssrc_p, sdst_p, weo_p, wcut_p, z, ng, ps)
        S = S.reshape(N_pad, F_pad)
        Q = Q.reshape(N_pad, F_pad)
        Mx = Mx.reshape(N_pad, F_pad)
        Mn = Mn.reshape(N_pad, F_pad)
        if cnt is None:
            cnt = S[:n, F_pad - 1]
            has = (cnt > 0)[:, None]
            denom = jnp.clip(cnt, 1.0)[:, None]
            avg_lin = jnp.mean(cnt)
            avg_log = jnp.mean(jnp.log(cnt + 1.0))
        out = _dense_post(h, a, S[:n, :F], Q[:n, :F], Mx[:n, :F], Mn[:n, :F],
                          cnt, has, denom, avg_lin, avg_log,
                          postW, postb, linW, linb)
        return out, cnt, has, denom, avg_lin, avg_log

    h1, cnt, has, denom, avg_lin, avg_log = run_conv(
        x, 3, 16, NP1, c1_pre_W, c1_pre_b, c1_post_W, c1_post_b,
        c1_lin_W, c1_lin_b)
    h1 = jax.nn.relu(h1)
    h2, *_ = run_conv(h1, 20, 32, NP2, c2_pre_W, c2_pre_b, c2_post_W,
                      c2_post_b, c2_lin_W, c2_lin_b, cnt, has, denom,
                      avg_lin, avg_log)
    h2 = jax.nn.relu(h2)
    return h2 @ out_W + out_b
